# baseline (speedup 1.0000x reference)
import functools

import jax
import jax.numpy as jnp
from jax import lax
from jax.experimental import pallas as pl
from jax.experimental.pallas import tpu as pltpu
from jax.experimental.pallas import tpu_sc as plsc

N = 4 * 8192
D = 512
NC, NS = 2, 16
NW = NC * NS
ROWS_PER_W = N // NW
CHUNK = 64
NCHUNK = ROWS_PER_W // CHUNK

_mesh = plsc.VectorSubcoreMesh(core_axis_name="c", subcore_axis_name="s")


@functools.partial(
    pl.kernel,
    mesh=_mesh,
    out_type=jax.ShapeDtypeStruct((N, D), jnp.float32),
    scratch_types=[
        pltpu.VMEM((ROWS_PER_W,), jnp.int32),
        pltpu.VMEM((CHUNK, D), jnp.float32),
        pltpu.VMEM((CHUNK, D), jnp.float32),
        pltpu.SemaphoreType.DMA,
        pltpu.SemaphoreType.DMA,
    ],
)
def _sc_embed(idx_hbm, table_hbm, out_hbm, idx_v, rows_a, rows_b, gsem, ssem):
    wid = lax.axis_index("s") * NC + lax.axis_index("c")
    base = wid * ROWS_PER_W
    pltpu.sync_copy(idx_hbm.at[pl.ds(base, ROWS_PER_W)], idx_v)

    bufs = (rows_a, rows_b)
    pending_g = pltpu.async_copy(
        table_hbm.at[idx_v.at[pl.ds(0, CHUNK)]], bufs[0], gsem)
    prev_s = None
    for ci in range(NCHUNK):
        buf = bufs[ci % 2]
        pending_g.wait()
        if prev_s is not None:
            prev_s.wait()
        if ci + 1 < NCHUNK:
            pending_g = pltpu.async_copy(
                table_hbm.at[idx_v.at[pl.ds((ci + 1) * CHUNK, CHUNK)]],
                bufs[(ci + 1) % 2], gsem)
        prev_s = pltpu.async_copy(
            buf, out_hbm.at[pl.ds(base + ci * CHUNK, CHUNK)], ssem)
    prev_s.wait()


def kernel(input, weight):
    idx = input.reshape(-1).astype(jnp.int32)
    out = _sc_embed(idx, weight)
    return out.reshape(input.shape + (weight.shape[1],))
